# Initial kernel scaffold; baseline (speedup 1.0000x reference)
import functools

import jax
import jax.numpy as jnp
from jax import lax
from jax.experimental import pallas as pl
from jax.experimental.pallas import tpu as pltpu
from jax.experimental.pallas import tpu_sc as plsc

F = 26
B = 16384
VOCAB = 100000
NC = 2
NS = 16
L = 16
NW = NC * NS
BPW = B // NW
CPW = BPW * F


def _body(xf_hbm, w_hbm, b_hbm, out_hbm, idx_v, g_v, acc_v, bias_v, sem):
    wid = lax.axis_index("s") * NC + lax.axis_index("c")
    base = wid * CPW

    pltpu.sync_copy(xf_hbm.at[pl.ds(base, CPW)], idx_v)
    pltpu.sync_copy(b_hbm, bias_v)

    iota = lax.iota(jnp.int32, L)

    def off_body(i, carry):
        j = i * L
        fld = lax.rem(j + iota, F)
        idx_v[pl.ds(j, L)] = idx_v[pl.ds(j, L)] + fld * VOCAB
        return carry

    lax.fori_loop(0, CPW // L, off_body, 0)

    pltpu.async_copy(w_hbm.at[idx_v], g_v, sem).wait()

    lane_f = iota * F
    zeros = jnp.zeros((L,), jnp.int32)
    bias_vec = bias_v[...]

    def red_body(t, carry):
        b0 = t * L

        def f_body(f, s):
            flat = lane_f + (b0 * F + f)
            return s + plsc.load_gather(g_v, [flat, zeros])

        s = lax.fori_loop(0, F, f_body, jnp.zeros((L,), jnp.float32))
        acc_v[pl.ds(b0, L)] = s / float(F) + bias_vec
        return carry

    lax.fori_loop(0, BPW // L, red_body, 0)

    pltpu.sync_copy(acc_v, out_hbm.at[pl.ds(wid * BPW, BPW)])


@jax.jit
def _emb(xf, w, b16):
    mesh = plsc.VectorSubcoreMesh(core_axis_name="c", subcore_axis_name="s")
    run = functools.partial(
        pl.kernel,
        mesh=mesh,
        out_type=jax.ShapeDtypeStruct((B,), jnp.float32),
        scratch_types=[
            pltpu.VMEM((CPW,), jnp.int32),
            pltpu.VMEM((CPW, 1), jnp.float32),
            pltpu.VMEM((BPW,), jnp.float32),
            pltpu.VMEM((L,), jnp.float32),
            pltpu.SemaphoreType.DMA,
        ],
    )(_body)
    return run(xf, w, b16)


def kernel(x, weight, bias):
    xf = x.astype(jnp.int32).reshape(-1)
    b16 = jnp.broadcast_to(bias.astype(jnp.float32), (L,))
    out = _emb(xf, weight, b16)
    return out.reshape(B, 1)


# speedup vs baseline: 1.1240x; 1.1240x over previous
import functools

import jax
import jax.numpy as jnp
from jax import lax
from jax.experimental import pallas as pl
from jax.experimental.pallas import tpu as pltpu
from jax.experimental.pallas import tpu_sc as plsc

F = 26
B = 16384
VOCAB = 100000
NC = 2
NS = 16
L = 16
NW = NC * NS
BPW = B // NW
CPW = BPW * F


def _body(xf_hbm, w_hbm, b_hbm, out_hbm, idx_v, g_v, acc_v, bias_v, sem):
    wid = lax.axis_index("s") * NC + lax.axis_index("c")
    base = wid * CPW

    pltpu.sync_copy(xf_hbm.at[pl.ds(base, CPW)], idx_v)
    pltpu.sync_copy(b_hbm, bias_v)

    iota = lax.iota(jnp.int32, L)

    def off_body(i, carry):
        j = i * L
        fld = lax.rem(j + iota, F)
        idx_v[pl.ds(j, L)] = idx_v[pl.ds(j, L)] + fld * VOCAB
        return carry

    lax.fori_loop(0, CPW // L, off_body, 0)

    pltpu.async_copy(w_hbm.at[idx_v], g_v, sem).wait()

    lane_f = iota * F
    bias_vec = bias_v[...]

    def red_body(t, carry):
        b0 = t * L

        def f_body(f, s):
            flat = lane_f + (b0 * F + f)
            return s + plsc.load_gather(g_v, [flat])

        s = lax.fori_loop(0, F, f_body, jnp.zeros((L,), jnp.float32))
        acc_v[pl.ds(b0, L)] = s / float(F) + bias_vec
        return carry

    lax.fori_loop(0, BPW // L, red_body, 0)

    pltpu.sync_copy(acc_v, out_hbm.at[pl.ds(wid * BPW, BPW)])


@jax.jit
def _emb(xf, w, b16):
    mesh = plsc.VectorSubcoreMesh(core_axis_name="c", subcore_axis_name="s")
    run = functools.partial(
        pl.kernel,
        mesh=mesh,
        out_type=jax.ShapeDtypeStruct((B,), jnp.float32),
        scratch_types=[
            pltpu.VMEM((CPW,), jnp.int32),
            pltpu.VMEM((CPW,), jnp.float32),
            pltpu.VMEM((BPW,), jnp.float32),
            pltpu.VMEM((L,), jnp.float32),
            pltpu.SemaphoreType.DMA,
        ],
        compiler_params=pltpu.CompilerParams(needs_layout_passes=False),
    )(_body)
    return run(xf, w, b16)


def kernel(x, weight, bias):
    xf = x.astype(jnp.int32).reshape(-1)
    b16 = jnp.broadcast_to(bias.astype(jnp.float32), (L,))
    out = _emb(xf, weight.reshape(-1), b16)
    return out.reshape(B, 1)


# speedup vs baseline: 2.6544x; 2.3615x over previous
import functools

import jax
import jax.numpy as jnp
from jax import lax
from jax.experimental import pallas as pl
from jax.experimental.pallas import tpu as pltpu
from jax.experimental.pallas import tpu_sc as plsc

F = 26
B = 16384
VOCAB = 100000
V = F * VOCAB
VMAIN = 2599936
NC = 2
NS = 16
L = 16
NW = NC * NS
BPW = B // NW
CPW = BPW * F


def _body(xf_hbm, wm_hbm, wt_hbm, b_hbm, out_hbm,
          idx_v, idc_v, g_v, acc_v, tail_v, bias_v, sem):
    wid = lax.axis_index("s") * NC + lax.axis_index("c")
    base = wid * CPW

    pltpu.sync_copy(xf_hbm.at[pl.ds(base, CPW)], idx_v)
    pltpu.sync_copy(wt_hbm, tail_v)
    pltpu.sync_copy(b_hbm, bias_v)

    iota = lax.iota(jnp.int32, L)

    def off_body(i, carry):
        j = i * L
        fld = lax.rem(j + iota, F)
        v = idx_v[pl.ds(j, L)] + fld * VOCAB
        idx_v[pl.ds(j, L)] = v
        idc_v[pl.ds(j, L)] = jnp.minimum(v, VMAIN - 1)
        return carry

    lax.fori_loop(0, CPW // L, off_body, 0)

    pltpu.async_copy(wm_hbm.at[idc_v], g_v, sem).wait()

    def fix_body(i, carry):
        j = i * L
        iv = idx_v[pl.ds(j, L)]
        m = iv >= VMAIN
        tpos = jnp.clip(iv - VMAIN, 0, V - VMAIN - 1)
        tv = plsc.load_gather(tail_v, [tpos])
        g_v[pl.ds(j, L)] = jnp.where(m, tv, g_v[pl.ds(j, L)])
        return carry

    lax.fori_loop(0, CPW // L, fix_body, 0)

    lane_f = iota * F
    bias_vec = bias_v[...]

    def red_body(t, carry):
        b0 = t * L

        def f_body(f, s):
            flat = lane_f + (b0 * F + f)
            return s + plsc.load_gather(g_v, [flat])

        s = lax.fori_loop(0, F, f_body, jnp.zeros((L,), jnp.float32))
        acc_v[pl.ds(b0, L)] = s / float(F) + bias_vec
        return carry

    lax.fori_loop(0, BPW // L, red_body, 0)

    pltpu.sync_copy(acc_v, out_hbm.at[pl.ds(wid * BPW, BPW)])


@jax.jit
def _emb(xf, wm, wt, b16):
    mesh = plsc.VectorSubcoreMesh(core_axis_name="c", subcore_axis_name="s")
    run = functools.partial(
        pl.kernel,
        mesh=mesh,
        out_type=jax.ShapeDtypeStruct((B,), jnp.float32),
        scratch_types=[
            pltpu.VMEM((CPW,), jnp.int32),
            pltpu.VMEM((CPW,), jnp.int32),
            pltpu.VMEM((CPW,), jnp.float32),
            pltpu.VMEM((BPW,), jnp.float32),
            pltpu.VMEM((V - VMAIN,), jnp.float32),
            pltpu.VMEM((L,), jnp.float32),
            pltpu.SemaphoreType.DMA,
        ],
        compiler_params=pltpu.CompilerParams(needs_layout_passes=False),
    )(_body)
    return run(xf, wm, wt, b16)


def kernel(x, weight, bias):
    xf = x.astype(jnp.int32).reshape(-1)
    b16 = jnp.broadcast_to(bias.astype(jnp.float32), (L,))
    w_main = weight[:VMAIN].reshape(-1)
    w_tail = weight[VMAIN:].reshape(-1)
    out = _emb(xf, w_main, w_tail, b16)
    return out.reshape(B, 1)


# speedup vs baseline: 2.8205x; 1.0626x over previous
import functools

import jax
import jax.numpy as jnp
from jax import lax
from jax.experimental import pallas as pl
from jax.experimental.pallas import tpu as pltpu
from jax.experimental.pallas import tpu_sc as plsc

F = 26
B = 16384
VOCAB = 100000
V = F * VOCAB
VMAIN = 2599936
NC = 2
NS = 16
L = 16
NW = NC * NS
BPW = B // NW
CPW = BPW * F

UN = 13
NCHUNK = 8
CHUNK = CPW // NCHUNK
BLOCKS_PER_CHUNK = CHUNK // (UN * L)

def _body(xf_hbm, wm_hbm, wt_hbm, b_hbm, out_hbm,
          idx_v, idc_v, g_v, acc_v, tail_v, bias_v, sem):
    wid = lax.axis_index("s") * NC + lax.axis_index("c")
    base = wid * CPW

    pltpu.sync_copy(xf_hbm.at[pl.ds(base, CPW)], idx_v)
    pltpu.sync_copy(wt_hbm, tail_v)
    pltpu.sync_copy(b_hbm, bias_v)

    iota = lax.iota(jnp.int32, L)
    fld = [lax.rem(u * L + iota, F) * VOCAB for u in range(UN)]

    copies = []
    for c in range(NCHUNK):

        def blk_body(bi, carry, c=c):
            j0 = c * CHUNK + bi * (UN * L)
            for u in range(UN):
                j = j0 + u * L
                v = idx_v[pl.ds(j, L)] + fld[u]
                idx_v[pl.ds(j, L)] = v
                idc_v[pl.ds(j, L)] = jnp.minimum(v, VMAIN - 1)
            return carry

        lax.fori_loop(0, BLOCKS_PER_CHUNK, blk_body, 0)
        copies.append(
            pltpu.async_copy(
                wm_hbm.at[idc_v.at[pl.ds(c * CHUNK, CHUNK)]],
                g_v.at[pl.ds(c * CHUNK, CHUNK)],
                sem,
            )
        )
    for cp in copies:
        cp.wait()

    def fix_body(i, carry):
        j0 = i * (4 * L)
        for u in range(4):
            j = j0 + u * L
            iv = idx_v[pl.ds(j, L)]
            m = iv >= VMAIN
            tpos = jnp.clip(iv - VMAIN, 0, V - VMAIN - 1)
            tv = plsc.load_gather(tail_v, [tpos])
            g_v[pl.ds(j, L)] = jnp.where(m, tv, g_v[pl.ds(j, L)])
        return carry

    lax.fori_loop(0, CPW // (4 * L), fix_body, 0)

    lane_f = iota * F
    bias_vec = bias_v[...]

    def red_body(t, carry):
        flat0 = lane_f + t * (L * F)
        s = plsc.load_gather(g_v, [flat0])
        for f in range(1, F):
            s = s + plsc.load_gather(g_v, [flat0 + f])
        acc_v[pl.ds(t * L, L)] = s / float(F) + bias_vec
        return carry

    lax.fori_loop(0, BPW // L, red_body, 0)

    pltpu.sync_copy(acc_v, out_hbm.at[pl.ds(wid * BPW, BPW)])


@jax.jit
def _emb(xf, wm, wt, b16):
    mesh = plsc.VectorSubcoreMesh(core_axis_name="c", subcore_axis_name="s")
    run = functools.partial(
        pl.kernel,
        mesh=mesh,
        out_type=jax.ShapeDtypeStruct((B,), jnp.float32),
        scratch_types=[
            pltpu.VMEM((CPW,), jnp.int32),
            pltpu.VMEM((CPW,), jnp.int32),
            pltpu.VMEM((CPW,), jnp.float32),
            pltpu.VMEM((BPW,), jnp.float32),
            pltpu.VMEM((V - VMAIN,), jnp.float32),
            pltpu.VMEM((L,), jnp.float32),
            pltpu.SemaphoreType.DMA,
        ],
        compiler_params=pltpu.CompilerParams(needs_layout_passes=False),
    )(_body)
    return run(xf, wm, wt, b16)


def kernel(x, weight, bias):
    xf = x.astype(jnp.int32).reshape(-1)
    b16 = jnp.broadcast_to(bias.astype(jnp.float32), (L,))
    w_main = weight[:VMAIN].reshape(-1)
    w_tail = weight[VMAIN:].reshape(-1)
    out = _emb(xf, w_main, w_tail, b16)
    return out.reshape(B, 1)


# speedup vs baseline: 3.0448x; 1.0795x over previous
import functools

import jax
import jax.numpy as jnp
from jax import lax
from jax.experimental import pallas as pl
from jax.experimental.pallas import tpu as pltpu
from jax.experimental.pallas import tpu_sc as plsc

F = 26
B = 16384
VOCAB = 100000
V = F * VOCAB
VMAIN = 2599936
XW = 128
NC = 2
NS = 16
L = 16
NW = NC * NS
BPW = B // NW
CPW = BPW * F

NCHUNK = 8
CHUNK = CPW // NCHUNK
SLICES = CPW // L
SPC = SLICES // NCHUNK
RPF = BPW // L


def _body(x_hbm, wm_hbm, wt_hbm, b_hbm, out_hbm,
          x_v, idc_v, g_v, acc_v, tail_v, bias_v, sem):
    wid = lax.axis_index("s") * NC + lax.axis_index("c")

    pltpu.sync_copy(x_hbm.at[pl.ds(wid * BPW, BPW), :], x_v)
    pltpu.sync_copy(wt_hbm, tail_v)
    pltpu.sync_copy(b_hbm, bias_v)

    iota = lax.iota(jnp.int32, L)
    rowsel = iota * XW

    copies = []
    for c in range(NCHUNK):

        def sl_body(i, carry, c=c):
            s = c * SPC + i
            f = s >> 5
            r0 = (s & 31) * L
            raw = plsc.load_gather(x_v, [r0 + iota, f + jnp.zeros((L,), jnp.int32)])
            idc_v[pl.ds(s * L, L)] = jnp.minimum(raw + f * VOCAB, VMAIN - 1)
            return carry

        lax.fori_loop(0, SPC, sl_body, 0)
        copies.append(
            pltpu.async_copy(
                wm_hbm.at[idc_v.at[pl.ds(c * CHUNK, CHUNK)]],
                g_v.at[pl.ds(c * CHUNK, CHUNK)],
                sem,
            )
        )
    for cp in copies:
        cp.wait()

    def fix_body(t, carry):
        r0 = t * L
        raw = plsc.load_gather(
            x_v, [r0 + iota, (F - 1) + jnp.zeros((L,), jnp.int32)]
        )
        iv = raw + (F - 1) * VOCAB
        m = iv >= VMAIN
        tpos = jnp.clip(iv - VMAIN, 0, V - VMAIN - 1)
        tv = plsc.load_gather(tail_v, [tpos])
        j = (F - 1) * BPW + r0
        g_v[pl.ds(j, L)] = jnp.where(m, tv, g_v[pl.ds(j, L)])
        return carry

    lax.fori_loop(0, RPF, fix_body, 0)

    bias_vec = bias_v[...]

    def red_body(t, carry):
        r0 = t * L
        s = g_v[pl.ds(r0, L)]
        for f in range(1, F):
            s = s + g_v[pl.ds(f * BPW + r0, L)]
        acc_v[pl.ds(r0, L)] = s / float(F) + bias_vec
        return carry

    lax.fori_loop(0, RPF, red_body, 0)

    pltpu.sync_copy(acc_v, out_hbm.at[pl.ds(wid * BPW, BPW)])


@jax.jit
def _emb(xp, wm, wt, b16):
    mesh = plsc.VectorSubcoreMesh(core_axis_name="c", subcore_axis_name="s")
    run = functools.partial(
        pl.kernel,
        mesh=mesh,
        out_type=jax.ShapeDtypeStruct((B,), jnp.float32),
        scratch_types=[
            pltpu.VMEM((BPW, XW), jnp.int32),
            pltpu.VMEM((CPW,), jnp.int32),
            pltpu.VMEM((CPW,), jnp.float32),
            pltpu.VMEM((BPW,), jnp.float32),
            pltpu.VMEM((V - VMAIN,), jnp.float32),
            pltpu.VMEM((L,), jnp.float32),
            pltpu.SemaphoreType.DMA,
        ],
        compiler_params=pltpu.CompilerParams(needs_layout_passes=False),
    )(_body)
    return run(xp, wm, wt, b16)


def kernel(x, weight, bias):
    xp = jnp.pad(x.astype(jnp.int32), ((0, 0), (0, XW - F)))
    b16 = jnp.broadcast_to(bias.astype(jnp.float32), (L,))
    w_main = weight[:VMAIN].reshape(-1)
    w_tail = weight[VMAIN:].reshape(-1)
    out = _emb(xp, w_main, w_tail, b16)
    return out.reshape(B, 1)


# speedup vs baseline: 3.3772x; 1.1092x over previous
import functools

import jax
import jax.numpy as jnp
from jax import lax
from jax.experimental import pallas as pl
from jax.experimental.pallas import tpu as pltpu
from jax.experimental.pallas import tpu_sc as plsc

F = 26
B = 16384
VOCAB = 100000
V = F * VOCAB
VMAIN = 2599936
NC = 2
NS = 16
L = 16
NW = NC * NS
BPW = B // NW
CPW = BPW * F

NCHUNK = 8
CHUNK = CPW // NCHUNK
SLICES = CPW // L
SPC = SLICES // NCHUNK
RPF = BPW // L


def _body(x_hbm, wm_hbm, wt_hbm, b_hbm, out_hbm,
          x_v, idc_v, g_v, acc_v, tail_v, bias_v, sem):
    wid = lax.axis_index("s") * NC + lax.axis_index("c")

    pltpu.sync_copy(x_hbm.at[pl.ds(wid * BPW, BPW), :], x_v)
    pltpu.sync_copy(wt_hbm, tail_v)
    pltpu.sync_copy(b_hbm, bias_v)

    iota = lax.iota(jnp.int32, L)

    copies = []
    for c in range(NCHUNK):

        def sl_body(i, carry, c=c):
            s = c * SPC + i
            f = s >> 5
            r0 = (s & 31) * L
            raw = plsc.load_gather(x_v, [r0 + iota, f + jnp.zeros((L,), jnp.int32)])
            idc_v[pl.ds(s * L, L)] = jnp.minimum(raw + f * VOCAB, VMAIN - 1)
            return carry

        lax.fori_loop(0, SPC, sl_body, 0)
        copies.append(
            pltpu.async_copy(
                wm_hbm.at[idc_v.at[pl.ds(c * CHUNK, CHUNK)]],
                g_v.at[pl.ds(c * CHUNK, CHUNK)],
                sem,
            )
        )
    for cp in copies:
        cp.wait()

    def fix_body(t, carry):
        r0 = t * L
        raw = plsc.load_gather(
            x_v, [r0 + iota, (F - 1) + jnp.zeros((L,), jnp.int32)]
        )
        iv = raw + (F - 1) * VOCAB
        m = iv >= VMAIN
        tpos = jnp.clip(iv - VMAIN, 0, V - VMAIN - 1)
        tv = plsc.load_gather(tail_v, [tpos])
        j = (F - 1) * BPW + r0
        g_v[pl.ds(j, L)] = jnp.where(m, tv, g_v[pl.ds(j, L)])
        return carry

    lax.fori_loop(0, RPF, fix_body, 0)

    bias_vec = bias_v[...]

    def red_body(t, carry):
        r0 = t * L
        s = g_v[pl.ds(r0, L)]
        for f in range(1, F):
            s = s + g_v[pl.ds(f * BPW + r0, L)]
        acc_v[pl.ds(r0, L)] = s / float(F) + bias_vec
        return carry

    lax.fori_loop(0, RPF, red_body, 0)

    pltpu.sync_copy(acc_v, out_hbm.at[pl.ds(wid * BPW, BPW)])


@jax.jit
def _emb(xp, wm, wt, b16):
    mesh = plsc.VectorSubcoreMesh(core_axis_name="c", subcore_axis_name="s")
    run = functools.partial(
        pl.kernel,
        mesh=mesh,
        out_type=jax.ShapeDtypeStruct((B,), jnp.float32),
        scratch_types=[
            pltpu.VMEM((BPW, F), jnp.int32),
            pltpu.VMEM((CPW,), jnp.int32),
            pltpu.VMEM((CPW,), jnp.float32),
            pltpu.VMEM((BPW,), jnp.float32),
            pltpu.VMEM((V - VMAIN,), jnp.float32),
            pltpu.VMEM((L,), jnp.float32),
            pltpu.SemaphoreType.DMA,
        ],
        compiler_params=pltpu.CompilerParams(needs_layout_passes=False),
    )(_body)
    return run(xp, wm, wt, b16)


def kernel(x, weight, bias):
    xp = x.astype(jnp.int32)
    b16 = jnp.broadcast_to(bias.astype(jnp.float32), (L,))
    w_main = weight[:VMAIN].reshape(-1)
    w_tail = weight[VMAIN:].reshape(-1)
    out = _emb(xp, w_main, w_tail, b16)
    return out.reshape(B, 1)
